# Initial kernel scaffold; baseline (speedup 1.0000x reference)
import functools

import jax
import jax.numpy as jnp
from jax import lax
from jax.experimental import pallas as pl
from jax.experimental.pallas import tpu as pltpu
from jax.experimental.pallas import tpu_sc as plsc

EMB = 32
B_ROWS = 4096
B_COLS = 200
B_TOTAL = B_ROWS * B_COLS
NUM_WORKERS = 32
B_PER_W = B_TOTAL // NUM_WORKERS
CHUNK = 128
N_CHUNKS = B_PER_W // CHUNK


def _emb_body(idx_hbm, table_hbm, out_hbm, idx_v, rows_v, sem):
    nc = 2
    wid = lax.axis_index("s") * nc + lax.axis_index("c")
    base_chunk = wid * N_CHUNKS
    pltpu.sync_copy(idx_hbm.at[pl.ds(base_chunk, N_CHUNKS)], idx_v)

    def step(j, carry):
        pltpu.async_copy(table_hbm.at[idx_v.at[j]], rows_v, sem).wait()
        pltpu.sync_copy(rows_v, out_hbm.at[pl.ds((base_chunk + j) * CHUNK, CHUNK)])
        return carry

    lax.fori_loop(0, N_CHUNKS, step, 0)


@functools.partial(
    pl.kernel,
    mesh=plsc.VectorSubcoreMesh(core_axis_name="c", subcore_axis_name="s"),
    out_type=jax.ShapeDtypeStruct((B_TOTAL, EMB), jnp.float32),
    scratch_types=[
        pltpu.VMEM((N_CHUNKS, CHUNK), jnp.int32),
        pltpu.VMEM((CHUNK, EMB), jnp.float32),
        pltpu.SemaphoreType.DMA,
    ],
)
def _emb_lookup(idx_hbm, table_hbm, out_hbm, idx_v, rows_v, sem):
    _emb_body(idx_hbm, table_hbm, out_hbm, idx_v, rows_v, sem)


def kernel(vocab_ids, node_embs_weight):
    idx = vocab_ids.reshape(B_TOTAL // CHUNK, CHUNK).astype(jnp.int32)
    out = _emb_lookup(idx, node_embs_weight)
    return out.reshape(B_ROWS, B_COLS, EMB)


# speedup vs baseline: 1.3067x; 1.3067x over previous
import functools

import jax
import jax.numpy as jnp
from jax import lax
from jax.experimental import pallas as pl
from jax.experimental.pallas import tpu as pltpu
from jax.experimental.pallas import tpu_sc as plsc

EMB = 32
B_ROWS = 4096
B_COLS = 200
B_TOTAL = B_ROWS * B_COLS
NUM_WORKERS = 32
B_PER_W = B_TOTAL // NUM_WORKERS
CHUNK = 128
N_CHUNKS = B_PER_W // CHUNK


def _emb_body(idx_hbm, table_hbm, out_hbm, idx_v, rows_v, sem):
    nc = 2
    wid = lax.axis_index("s") * nc + lax.axis_index("c")
    base_chunk = wid * N_CHUNKS
    pltpu.sync_copy(idx_hbm.at[pl.ds(base_chunk, N_CHUNKS)], idx_v)

    def step(j, carry):
        pltpu.async_copy(table_hbm.at[idx_v.at[j]], rows_v, sem).wait()
        pltpu.sync_copy(rows_v, out_hbm.at[pl.ds((base_chunk + j) * CHUNK, CHUNK)])
        return carry

    lax.fori_loop(0, N_CHUNKS, step, 0)


@functools.partial(
    pl.kernel,
    mesh=plsc.VectorSubcoreMesh(core_axis_name="c", subcore_axis_name="s"),
    out_type=jax.ShapeDtypeStruct((B_TOTAL, EMB), jnp.float32),
    scratch_types=[
        pltpu.VMEM((N_CHUNKS, CHUNK), jnp.int32),
        pltpu.VMEM((CHUNK, EMB), jnp.float32),
        pltpu.SemaphoreType.DMA,
    ],
    compiler_params=pltpu.CompilerParams(use_tc_tiling_on_sc=False),
)
def _emb_lookup(idx_hbm, table_hbm, out_hbm, idx_v, rows_v, sem):
    _emb_body(idx_hbm, table_hbm, out_hbm, idx_v, rows_v, sem)


def kernel(vocab_ids, node_embs_weight):
    idx = vocab_ids.reshape(B_TOTAL // CHUNK, CHUNK).astype(jnp.int32)
    out = _emb_lookup(idx, node_embs_weight)
    return out.reshape(B_ROWS, B_COLS, EMB)


# speedup vs baseline: 1.4483x; 1.1083x over previous
import functools

import jax
import jax.numpy as jnp
from jax import lax
from jax.experimental import pallas as pl
from jax.experimental.pallas import tpu as pltpu
from jax.experimental.pallas import tpu_sc as plsc

EMB = 32
B_ROWS = 4096
B_COLS = 200
B_TOTAL = B_ROWS * B_COLS
NUM_WORKERS = 32
B_PER_W = B_TOTAL // NUM_WORKERS
CHUNK = 512
N_CHUNKS = B_PER_W // CHUNK


def _emb_body(idx_hbm, table_hbm, out_hbm, idx_v, rows_v, sem):
    nc = 2
    wid = lax.axis_index("s") * nc + lax.axis_index("c")
    base_chunk = wid * N_CHUNKS
    pltpu.sync_copy(idx_hbm.at[pl.ds(base_chunk, N_CHUNKS)], idx_v)

    def step(j, carry):
        pltpu.async_copy(table_hbm.at[idx_v.at[j]], rows_v, sem).wait()
        pltpu.sync_copy(rows_v, out_hbm.at[pl.ds((base_chunk + j) * CHUNK, CHUNK)])
        return carry

    lax.fori_loop(0, N_CHUNKS, step, 0)


@functools.partial(
    pl.kernel,
    mesh=plsc.VectorSubcoreMesh(core_axis_name="c", subcore_axis_name="s"),
    out_type=jax.ShapeDtypeStruct((B_TOTAL, EMB), jnp.float32),
    scratch_types=[
        pltpu.VMEM((N_CHUNKS, CHUNK), jnp.int32),
        pltpu.VMEM((CHUNK, EMB), jnp.float32),
        pltpu.SemaphoreType.DMA,
    ],
    compiler_params=pltpu.CompilerParams(use_tc_tiling_on_sc=False),
)
def _emb_lookup(idx_hbm, table_hbm, out_hbm, idx_v, rows_v, sem):
    _emb_body(idx_hbm, table_hbm, out_hbm, idx_v, rows_v, sem)


def kernel(vocab_ids, node_embs_weight):
    idx = vocab_ids.reshape(B_TOTAL // CHUNK, CHUNK).astype(jnp.int32)
    out = _emb_lookup(idx, node_embs_weight)
    return out.reshape(B_ROWS, B_COLS, EMB)


# speedup vs baseline: 1.4982x; 1.0345x over previous
import functools

import jax
import jax.numpy as jnp
from jax import lax
from jax.experimental import pallas as pl
from jax.experimental.pallas import tpu as pltpu
from jax.experimental.pallas import tpu_sc as plsc

EMB = 32
B_ROWS = 4096
B_COLS = 200
B_TOTAL = B_ROWS * B_COLS
NUM_WORKERS = 32
B_PER_W = B_TOTAL // NUM_WORKERS
CHUNK = 512
N_CHUNKS = B_PER_W // CHUNK


def _emb_body(idx_hbm, table_hbm, out_hbm, idx_v, rows_v, gsem0, gsem1, ssem0, ssem1):
    nc = 2
    wid = lax.axis_index("s") * nc + lax.axis_index("c")
    base_chunk = wid * N_CHUNKS
    gsems = (gsem0, gsem1)
    ssems = (ssem0, ssem1)

    pltpu.sync_copy(idx_hbm.at[pl.ds(base_chunk, N_CHUNKS)], idx_v)

    def fire_gather(j, p):
        pltpu.async_copy(table_hbm.at[idx_v.at[j]], rows_v.at[p], gsems[p])

    def wait_gather(p):
        pltpu.make_async_copy(table_hbm.at[idx_v.at[0]], rows_v.at[p], gsems[p]).wait()

    def fire_store(j, p):
        pltpu.async_copy(
            rows_v.at[p], out_hbm.at[pl.ds((base_chunk + j) * CHUNK, CHUNK)], ssems[p]
        )

    def wait_store(p):
        pltpu.make_async_copy(
            rows_v.at[p], out_hbm.at[pl.ds(base_chunk * CHUNK, CHUNK)], ssems[p]
        ).wait()

    fire_gather(0, 0)

    def group(g, carry):
        for p in (0, 1):
            j = 2 * g + p
            q = 1 - p

            @pl.when(j >= 1)
            def _():
                wait_store(q)

            @pl.when(j + 1 < N_CHUNKS)
            def _():
                fire_gather(j + 1, q)

            wait_gather(p)
            fire_store(j, p)
        return carry

    lax.fori_loop(0, N_CHUNKS // 2, group, 0)
    wait_store((N_CHUNKS - 1) % 2)


@functools.partial(
    pl.kernel,
    mesh=plsc.VectorSubcoreMesh(core_axis_name="c", subcore_axis_name="s"),
    out_type=jax.ShapeDtypeStruct((B_TOTAL, EMB), jnp.float32),
    scratch_types=[
        pltpu.VMEM((N_CHUNKS, CHUNK), jnp.int32),
        pltpu.VMEM((2, CHUNK, EMB), jnp.float32),
        pltpu.SemaphoreType.DMA,
        pltpu.SemaphoreType.DMA,
        pltpu.SemaphoreType.DMA,
        pltpu.SemaphoreType.DMA,
    ],
    compiler_params=pltpu.CompilerParams(use_tc_tiling_on_sc=False),
)
def _emb_lookup(idx_hbm, table_hbm, out_hbm, idx_v, rows_v, gsem0, gsem1, ssem0, ssem1):
    _emb_body(idx_hbm, table_hbm, out_hbm, idx_v, rows_v, gsem0, gsem1, ssem0, ssem1)


def kernel(vocab_ids, node_embs_weight):
    idx = vocab_ids.reshape(B_TOTAL // CHUNK, CHUNK).astype(jnp.int32)
    out = _emb_lookup(idx, node_embs_weight)
    return out.reshape(B_ROWS, B_COLS, EMB)
